# baseline (speedup 1.0000x reference)
import functools

import jax
import jax.numpy as jnp
from jax import lax
from jax.experimental import pallas as pl
from jax.experimental.pallas import tpu as pltpu
from jax.experimental.pallas import tpu_sc as plsc

VOCAB = 100000
Z_DIM = 32
BATCH = 1024
VPAD = ((VOCAB + 127) // 128) * 128
KA = 48
MB = 32
NG = BATCH // MB


def _gather_sc(table, idx):
    info = plsc.get_sparse_core_info()
    nc, ns = info.num_cores, info.num_subcores
    nw = nc * ns
    bpw = BATCH // nw
    mesh = plsc.VectorSubcoreMesh(core_axis_name="c", subcore_axis_name="s")

    @functools.partial(
        pl.kernel,
        mesh=mesh,
        out_type=jax.ShapeDtypeStruct((BATCH, Z_DIM), jnp.float32),
        scratch_types=[
            pltpu.VMEM((bpw,), jnp.int32),
            pltpu.VMEM((bpw, Z_DIM), jnp.float32),
            pltpu.SemaphoreType.DMA,
        ],
        compiler_params=pltpu.CompilerParams(use_tc_tiling_on_sc=False),
    )
    def gather(table_hbm, idx_hbm, out_hbm, idx_v, rows_v, sem):
        wid = lax.axis_index("s") * nc + lax.axis_index("c")
        base = wid * bpw
        pltpu.sync_copy(idx_hbm.at[pl.ds(base, bpw)], idx_v)
        pltpu.async_copy(table_hbm.at[idx_v], rows_v, sem).wait()
        pltpu.sync_copy(rows_v, out_hbm.at[pl.ds(base, bpw)])

    return gather(table, idx)


def _band_body(emb_ref, wt_ref, out_ref):
    sc = lax.dot_general(
        emb_ref[...], wt_ref[...], (((1,), (0,)), ((), ())),
        preferred_element_type=jnp.float32,
    )
    m = jnp.max(sc, axis=1, keepdims=True)
    s = jnp.sum(jnp.exp(sc - m), axis=1, keepdims=True)
    out_ref[...] = (sc - (m + jnp.log(s))).astype(jnp.bfloat16)


def _fused_logsoftmax(emb_aug, wt_aug):
    return pl.pallas_call(
        _band_body,
        grid=(NG,),
        in_specs=[
            pl.BlockSpec((MB, KA), lambda g: (g, 0)),
            pl.BlockSpec((KA, VPAD), lambda g: (0, 0)),
        ],
        out_specs=pl.BlockSpec((MB, VPAD), lambda g: (g, 0)),
        out_shape=jax.ShapeDtypeStruct((BATCH, VPAD), jnp.bfloat16),
        compiler_params=pltpu.CompilerParams(
            vmem_limit_bytes=100 * 1024 * 1024),
    )(emb_aug, wt_aug)


def kernel(input_word, emb_table, W_out, b_out):
    idx = input_word.astype(jnp.int32)
    emb = _gather_sc(emb_table, idx)
    emb_aug = jnp.concatenate(
        [emb.astype(jnp.bfloat16),
         jnp.ones((BATCH, 1), jnp.bfloat16),
         jnp.zeros((BATCH, KA - Z_DIM - 1), jnp.bfloat16)], axis=1)
    wt_aug = jnp.concatenate(
        [jnp.pad(W_out.astype(jnp.bfloat16).T, ((0, 0), (0, VPAD - VOCAB))),
         jnp.pad(b_out, (0, VPAD - VOCAB),
                 constant_values=-1e30).reshape(1, VPAD).astype(jnp.bfloat16),
         jnp.zeros((KA - Z_DIM - 1, VPAD), jnp.bfloat16)], axis=0)
    out16 = _fused_logsoftmax(emb_aug, wt_aug)
    return out16[:, :VOCAB].astype(jnp.float32)
